# Initial kernel scaffold; baseline (speedup 1.0000x reference)
import jax
import jax.numpy as jnp
from jax.experimental import pallas as pl


def kernel(pos, batch, weight):
    raise NotImplementedError("write your pallas kernel here")


# speedup vs baseline: 2.3049x; 2.3049x over previous
import functools

import jax
import jax.numpy as jnp
from jax import lax
from jax.experimental import pallas as pl
from jax.experimental.pallas import tpu as pltpu
from jax.experimental.pallas import tpu_sc as plsc

N = 3200000
S = 1024
EPS = 1e-5
L = 16
NC, NS = 2, 16
NW = NC * NS
PER_W = N // NW
CH = 4000
NCH = PER_W // CH
VPC = CH // L

_mesh = plsc.VectorSubcoreMesh(core_axis_name="c", subcore_axis_name="s")
_params = pltpu.CompilerParams(needs_layout_passes=False)


def _fast_norm(n2):
    i = lax.bitcast_convert_type(n2, jnp.int32)
    i = jnp.full((L,), 0x5F3759DF, jnp.int32) - lax.shift_right_logical(i, 1)
    y = lax.bitcast_convert_type(i, jnp.float32)
    ah = n2 * jnp.full((L,), 0.5, jnp.float32)
    c15 = jnp.full((L,), 1.5, jnp.float32)
    y = y * (c15 - ah * y * y)
    y = y * (c15 - ah * y * y)
    return n2 * y


@functools.partial(
    pl.kernel,
    mesh=_mesh,
    out_type=[
        jax.ShapeDtypeStruct((NW, S), jnp.float32),
        jax.ShapeDtypeStruct((NW, S), jnp.float32),
    ],
    scratch_types=[
        pltpu.VMEM((CH * 3,), jnp.float32),
        pltpu.VMEM((CH,), jnp.int32),
        pltpu.VMEM((L * S,), jnp.float32),
        pltpu.VMEM((L * S,), jnp.float32),
        pltpu.VMEM((S,), jnp.float32),
        pltpu.VMEM((S,), jnp.float32),
    ],
    compiler_params=_params,
)
def _pass1(pos_hbm, batch_hbm, psum_hbm, pcnt_hbm,
           pbuf, bbuf, accs, accc, reds, redc):
    wid = lax.axis_index("s") * NC + lax.axis_index("c")
    base_w = wid * PER_W
    lanes = lax.iota(jnp.int32, L)
    lanes3 = lanes * 3
    laneoff = lanes * S
    zero = jnp.zeros((L,), jnp.float32)
    ones = jnp.ones((L,), jnp.float32)
    one_i = jnp.full((L,), 1, jnp.int32)

    def zero_body(i, carry):
        accs[pl.ds(i * L, L)] = zero
        accc[pl.ds(i * L, L)] = zero
        return carry

    lax.fori_loop(0, S, zero_body, 0)

    def chunk_body(k, carry):
        base = base_w + k * CH
        pltpu.sync_copy(pos_hbm.at[pl.ds(base * 3, CH * 3)], pbuf)
        pltpu.sync_copy(batch_hbm.at[pl.ds(base, CH)], bbuf)

        def vec_body(v, c2):
            ix = lanes3 + v * (3 * L)
            iy = ix + one_i
            iz = iy + one_i
            b = bbuf[pl.ds(v * L, L)]
            x = plsc.load_gather(pbuf, [ix])
            y = plsc.load_gather(pbuf, [iy])
            z = plsc.load_gather(pbuf, [iz])
            nrm = _fast_norm(x * x + y * y + z * z)
            idx = b + laneoff
            plsc.addupdate_scatter(accs, [idx], nrm)
            plsc.addupdate_scatter(accc, [idx], ones)
            return c2

        lax.fori_loop(0, VPC, vec_body, 0)
        return carry

    lax.fori_loop(0, NCH, chunk_body, 0)

    def red_body(g, carry):
        sbase = g * L
        ssum = accs[pl.ds(sbase, L)]
        scnt = accc[pl.ds(sbase, L)]
        for c in range(1, L):
            ssum = ssum + accs[pl.ds(c * S + sbase, L)]
            scnt = scnt + accc[pl.ds(c * S + sbase, L)]
        reds[pl.ds(sbase, L)] = ssum
        redc[pl.ds(sbase, L)] = scnt
        return carry

    lax.fori_loop(0, S // L, red_body, 0)
    pltpu.sync_copy(reds, psum_hbm.at[wid])
    pltpu.sync_copy(redc, pcnt_hbm.at[wid])


@functools.partial(
    pl.kernel,
    mesh=_mesh,
    out_type=jax.ShapeDtypeStruct((N * 3,), jnp.float32),
    scratch_types=[
        pltpu.VMEM((NW, S), jnp.float32),
        pltpu.VMEM((NW, S), jnp.float32),
        pltpu.VMEM((S,), jnp.float32),
        pltpu.VMEM((L,), jnp.float32),
        pltpu.VMEM((CH * 3,), jnp.float32),
        pltpu.VMEM((CH,), jnp.int32),
        pltpu.VMEM((CH * 3,), jnp.float32),
    ],
    compiler_params=_params,
)
def _pass2(pos_hbm, batch_hbm, w_hbm, psum_hbm, pcnt_hbm, out_hbm,
           psv, pcv, rbuf, wbuf, pbuf, bbuf, obuf):
    wid = lax.axis_index("s") * NC + lax.axis_index("c")
    base_w = wid * PER_W
    lanes = lax.iota(jnp.int32, L)
    lanes3 = lanes * 3
    one_i = jnp.full((L,), 1, jnp.int32)
    onev = jnp.ones((L,), jnp.float32)
    epsv = jnp.full((L,), EPS, jnp.float32)

    pltpu.sync_copy(psum_hbm, psv)
    pltpu.sync_copy(pcnt_hbm, pcv)
    pltpu.sync_copy(w_hbm, wbuf)
    w = wbuf[pl.ds(0, L)]

    def r_body(g, carry):
        sbase = g * L
        ssum = psv[0, pl.ds(sbase, L)]
        scnt = pcv[0, pl.ds(sbase, L)]
        for t in range(1, NW):
            ssum = ssum + psv[t, pl.ds(sbase, L)]
            scnt = scnt + pcv[t, pl.ds(sbase, L)]
        mean = ssum / jnp.maximum(scnt, onev)
        rbuf[pl.ds(sbase, L)] = w / (mean + epsv)
        return carry

    lax.fori_loop(0, S // L, r_body, 0)

    def chunk_body(k, carry):
        base = base_w + k * CH
        pltpu.sync_copy(pos_hbm.at[pl.ds(base * 3, CH * 3)], pbuf)
        pltpu.sync_copy(batch_hbm.at[pl.ds(base, CH)], bbuf)

        def vec_body(v, c2):
            ix = lanes3 + v * (3 * L)
            iy = ix + one_i
            iz = iy + one_i
            b = bbuf[pl.ds(v * L, L)]
            r = plsc.load_gather(rbuf, [b])
            x = plsc.load_gather(pbuf, [ix]) * r
            y = plsc.load_gather(pbuf, [iy]) * r
            z = plsc.load_gather(pbuf, [iz]) * r
            plsc.store_scatter(obuf, [ix], x)
            plsc.store_scatter(obuf, [iy], y)
            plsc.store_scatter(obuf, [iz], z)
            return c2

        lax.fori_loop(0, VPC, vec_body, 0)
        pltpu.sync_copy(obuf, out_hbm.at[pl.ds(base * 3, CH * 3)])
        return carry

    lax.fori_loop(0, NCH, chunk_body, 0)


def kernel(pos, batch, weight):
    pos_flat = pos.reshape(N * 3)
    wvec = jnp.broadcast_to(weight.reshape(1), (L,)).astype(jnp.float32)
    psum, pcnt = _pass1(pos_flat, batch)
    out_flat = _pass2(pos_flat, batch, wvec, psum, pcnt)
    return out_flat.reshape(N, 3)


# speedup vs baseline: 15.5211x; 6.7339x over previous
import functools

import jax
import jax.numpy as jnp
from jax import lax
from jax.experimental import pallas as pl
from jax.experimental.pallas import tpu as pltpu
from jax.experimental.pallas import tpu_sc as plsc

N = 3200000
S = 1024
EPS = 1e-5
L = 16
NC, NS = 2, 16
NW = NC * NS
PER_W = N // NW
CH = 4000
NCH = PER_W // CH
VPC = CH // L

_mesh = plsc.VectorSubcoreMesh(core_axis_name="c", subcore_axis_name="s")
_params = pltpu.CompilerParams(needs_layout_passes=False)


def _fast_norm(n2):
    i = lax.bitcast_convert_type(n2, jnp.int32)
    i = jnp.full((L,), 0x5F3759DF, jnp.int32) - lax.shift_right_logical(i, 1)
    y = lax.bitcast_convert_type(i, jnp.float32)
    ah = n2 * jnp.full((L,), 0.5, jnp.float32)
    c15 = jnp.full((L,), 1.5, jnp.float32)
    y = y * (c15 - ah * y * y)
    y = y * (c15 - ah * y * y)
    return n2 * y


@functools.partial(
    pl.kernel,
    mesh=_mesh,
    out_type=[
        jax.ShapeDtypeStruct((NW, S), jnp.float32),
        jax.ShapeDtypeStruct((NW, S), jnp.float32),
    ],
    scratch_types=[
        pltpu.VMEM((CH,), jnp.float32),
        pltpu.VMEM((CH,), jnp.float32),
        pltpu.VMEM((CH,), jnp.float32),
        pltpu.VMEM((CH,), jnp.int32),
        pltpu.VMEM((L * S,), jnp.float32),
        pltpu.VMEM((L * S,), jnp.float32),
        pltpu.VMEM((S,), jnp.float32),
        pltpu.VMEM((S,), jnp.float32),
    ],
    compiler_params=_params,
)
def _pass1(pos_hbm, batch_hbm, psum_hbm, pcnt_hbm,
           xbuf, ybuf, zbuf, bbuf, accs, accc, reds, redc):
    wid = lax.axis_index("s") * NC + lax.axis_index("c")
    base_w = wid * PER_W
    lanes = lax.iota(jnp.int32, L)
    laneoff = lanes * S
    zero = jnp.zeros((L,), jnp.float32)
    ones = jnp.ones((L,), jnp.float32)

    def zero_body(i, carry):
        accs[pl.ds(i * L, L)] = zero
        accc[pl.ds(i * L, L)] = zero
        return carry

    lax.fori_loop(0, S, zero_body, 0)

    def chunk_body(k, carry):
        base = base_w + k * CH
        pltpu.sync_copy(pos_hbm.at[pl.ds(base, CH)], xbuf)
        pltpu.sync_copy(pos_hbm.at[pl.ds(N + base, CH)], ybuf)
        pltpu.sync_copy(pos_hbm.at[pl.ds(2 * N + base, CH)], zbuf)
        pltpu.sync_copy(batch_hbm.at[pl.ds(base, CH)], bbuf)

        def vec_body(v, c2):
            o = v * L
            b = bbuf[pl.ds(o, L)]
            x = xbuf[pl.ds(o, L)]
            y = ybuf[pl.ds(o, L)]
            z = zbuf[pl.ds(o, L)]
            nrm = _fast_norm(x * x + y * y + z * z)
            idx = b + laneoff
            plsc.addupdate_scatter(accs, [idx], nrm)
            plsc.addupdate_scatter(accc, [idx], ones)
            return c2

        lax.fori_loop(0, VPC, vec_body, 0)
        return carry

    lax.fori_loop(0, NCH, chunk_body, 0)

    def red_body(g, carry):
        sbase = g * L
        ssum = accs[pl.ds(sbase, L)]
        scnt = accc[pl.ds(sbase, L)]
        for c in range(1, L):
            ssum = ssum + accs[pl.ds(c * S + sbase, L)]
            scnt = scnt + accc[pl.ds(c * S + sbase, L)]
        reds[pl.ds(sbase, L)] = ssum
        redc[pl.ds(sbase, L)] = scnt
        return carry

    lax.fori_loop(0, S // L, red_body, 0)
    pltpu.sync_copy(reds, psum_hbm.at[wid])
    pltpu.sync_copy(redc, pcnt_hbm.at[wid])


@functools.partial(
    pl.kernel,
    mesh=_mesh,
    out_type=jax.ShapeDtypeStruct((3 * N,), jnp.float32),
    scratch_types=[
        pltpu.VMEM((NW, S), jnp.float32),
        pltpu.VMEM((NW, S), jnp.float32),
        pltpu.VMEM((S,), jnp.float32),
        pltpu.VMEM((L,), jnp.float32),
        pltpu.VMEM((CH,), jnp.float32),
        pltpu.VMEM((CH,), jnp.float32),
        pltpu.VMEM((CH,), jnp.float32),
        pltpu.VMEM((CH,), jnp.int32),
        pltpu.VMEM((CH,), jnp.float32),
        pltpu.VMEM((CH,), jnp.float32),
        pltpu.VMEM((CH,), jnp.float32),
    ],
    compiler_params=_params,
)
def _pass2(pos_hbm, batch_hbm, w_hbm, psum_hbm, pcnt_hbm, out_hbm,
           psv, pcv, rbuf, wbuf, xbuf, ybuf, zbuf, bbuf, xout, yout, zout):
    wid = lax.axis_index("s") * NC + lax.axis_index("c")
    base_w = wid * PER_W
    onev = jnp.ones((L,), jnp.float32)
    epsv = jnp.full((L,), EPS, jnp.float32)

    pltpu.sync_copy(psum_hbm, psv)
    pltpu.sync_copy(pcnt_hbm, pcv)
    pltpu.sync_copy(w_hbm, wbuf)
    w = wbuf[pl.ds(0, L)]

    def r_body(g, carry):
        sbase = g * L
        ssum = psv[0, pl.ds(sbase, L)]
        scnt = pcv[0, pl.ds(sbase, L)]
        for t in range(1, NW):
            ssum = ssum + psv[t, pl.ds(sbase, L)]
            scnt = scnt + pcv[t, pl.ds(sbase, L)]
        mean = ssum / jnp.maximum(scnt, onev)
        rbuf[pl.ds(sbase, L)] = w / (mean + epsv)
        return carry

    lax.fori_loop(0, S // L, r_body, 0)

    def chunk_body(k, carry):
        base = base_w + k * CH
        pltpu.sync_copy(pos_hbm.at[pl.ds(base, CH)], xbuf)
        pltpu.sync_copy(pos_hbm.at[pl.ds(N + base, CH)], ybuf)
        pltpu.sync_copy(pos_hbm.at[pl.ds(2 * N + base, CH)], zbuf)
        pltpu.sync_copy(batch_hbm.at[pl.ds(base, CH)], bbuf)

        def vec_body(v, c2):
            o = v * L
            b = bbuf[pl.ds(o, L)]
            r = plsc.load_gather(rbuf, [b])
            xout[pl.ds(o, L)] = xbuf[pl.ds(o, L)] * r
            yout[pl.ds(o, L)] = ybuf[pl.ds(o, L)] * r
            zout[pl.ds(o, L)] = zbuf[pl.ds(o, L)] * r
            return c2

        lax.fori_loop(0, VPC, vec_body, 0)
        pltpu.sync_copy(xout, out_hbm.at[pl.ds(base, CH)])
        pltpu.sync_copy(yout, out_hbm.at[pl.ds(N + base, CH)])
        pltpu.sync_copy(zout, out_hbm.at[pl.ds(2 * N + base, CH)])
        return carry

    lax.fori_loop(0, NCH, chunk_body, 0)


def kernel(pos, batch, weight):
    pos_flat = jnp.swapaxes(pos, 0, 1).reshape(3 * N)
    wvec = jnp.broadcast_to(weight.reshape(1), (L,)).astype(jnp.float32)
    psum, pcnt = _pass1(pos_flat, batch)
    out_flat = _pass2(pos_flat, batch, wvec, psum, pcnt)
    return jnp.swapaxes(out_flat.reshape(3, N), 0, 1)


# speedup vs baseline: 60.1261x; 3.8738x over previous
import functools

import jax
import jax.numpy as jnp
from jax import lax
from jax.experimental import pallas as pl
from jax.experimental.pallas import tpu as pltpu
from jax.experimental.pallas import tpu_sc as plsc

N = 3200000
S = 1024
EPS = 1e-5
L = 16
NC, NS = 2, 16
NW = NC * NS
CHB = 3200
NCHT = N // CHB
VPC = CHB // L

_mesh = plsc.VectorSubcoreMesh(core_axis_name="c", subcore_axis_name="s")
_params = pltpu.CompilerParams(needs_layout_passes=False)


def _fast_norm(n2):
    i = lax.bitcast_convert_type(n2, jnp.int32)
    i = jnp.full((L,), 0x5F3759DF, jnp.int32) - lax.shift_right_logical(i, 1)
    y = lax.bitcast_convert_type(i, jnp.float32)
    ah = n2 * jnp.full((L,), 0.5, jnp.float32)
    c15 = jnp.full((L,), 1.5, jnp.float32)
    y = y * (c15 - ah * y * y)
    y = y * (c15 - ah * y * y)
    return n2 * y


def _n_chunks(wid):
    return jnp.int32(NCHT // NW) + jnp.where(wid < NCHT % NW, 1, 0).astype(jnp.int32)


@functools.partial(
    pl.kernel,
    mesh=_mesh,
    out_type=[
        jax.ShapeDtypeStruct((NW, S), jnp.float32),
        jax.ShapeDtypeStruct((NW, S), jnp.float32),
    ],
    scratch_types=[
        pltpu.VMEM((3, CHB), jnp.float32),
        pltpu.VMEM((CHB,), jnp.int32),
        pltpu.VMEM((L * S,), jnp.float32),
        pltpu.VMEM((L * S,), jnp.float32),
        pltpu.VMEM((S,), jnp.float32),
        pltpu.VMEM((S,), jnp.float32),
    ],
    compiler_params=_params,
)
def _pass1(pos_hbm, batch_hbm, psum_hbm, pcnt_hbm,
           buf, bbuf, accs, accc, reds, redc):
    wid = lax.axis_index("s") * NC + lax.axis_index("c")
    lanes = lax.iota(jnp.int32, L)
    laneoff = lanes * S
    zero = jnp.zeros((L,), jnp.float32)
    ones = jnp.ones((L,), jnp.float32)

    def zero_body(i, carry):
        accs[pl.ds(i * L, L)] = zero
        accc[pl.ds(i * L, L)] = zero
        return carry

    lax.fori_loop(0, S, zero_body, 0)

    def chunk_body(k, carry):
        base = (wid + k * NW) * CHB
        pltpu.sync_copy(pos_hbm.at[:, pl.ds(base, CHB)], buf)
        pltpu.sync_copy(batch_hbm.at[pl.ds(base, CHB)], bbuf)

        def vec_body(v, c2):
            o = v * L
            b = bbuf[pl.ds(o, L)]
            x = buf[0, pl.ds(o, L)]
            y = buf[1, pl.ds(o, L)]
            z = buf[2, pl.ds(o, L)]
            nrm = _fast_norm(x * x + y * y + z * z)
            idx = b + laneoff
            plsc.addupdate_scatter(accs, [idx], nrm)
            plsc.addupdate_scatter(accc, [idx], ones)
            return c2

        lax.fori_loop(0, VPC, vec_body, 0)
        return carry

    lax.fori_loop(0, _n_chunks(wid), chunk_body, 0)

    def red_body(g, carry):
        sbase = g * L
        ssum = accs[pl.ds(sbase, L)]
        scnt = accc[pl.ds(sbase, L)]
        for c in range(1, L):
            ssum = ssum + accs[pl.ds(c * S + sbase, L)]
            scnt = scnt + accc[pl.ds(c * S + sbase, L)]
        reds[pl.ds(sbase, L)] = ssum
        redc[pl.ds(sbase, L)] = scnt
        return carry

    lax.fori_loop(0, S // L, red_body, 0)
    pltpu.sync_copy(reds, psum_hbm.at[wid])
    pltpu.sync_copy(redc, pcnt_hbm.at[wid])


@functools.partial(
    pl.kernel,
    mesh=_mesh,
    out_type=jax.ShapeDtypeStruct((3, N), jnp.float32),
    scratch_types=[
        pltpu.VMEM((NW, S), jnp.float32),
        pltpu.VMEM((NW, S), jnp.float32),
        pltpu.VMEM((S,), jnp.float32),
        pltpu.VMEM((L,), jnp.float32),
        pltpu.VMEM((3, CHB), jnp.float32),
        pltpu.VMEM((CHB,), jnp.int32),
        pltpu.VMEM((3, CHB), jnp.float32),
    ],
    compiler_params=_params,
)
def _pass2(pos_hbm, batch_hbm, w_hbm, psum_hbm, pcnt_hbm, out_hbm,
           psv, pcv, rbuf, wbuf, buf, bbuf, obuf):
    wid = lax.axis_index("s") * NC + lax.axis_index("c")
    onev = jnp.ones((L,), jnp.float32)
    epsv = jnp.full((L,), EPS, jnp.float32)

    pltpu.sync_copy(psum_hbm, psv)
    pltpu.sync_copy(pcnt_hbm, pcv)
    pltpu.sync_copy(w_hbm, wbuf)
    w = wbuf[pl.ds(0, L)]

    def r_body(g, carry):
        sbase = g * L
        ssum = psv[0, pl.ds(sbase, L)]
        scnt = pcv[0, pl.ds(sbase, L)]
        for t in range(1, NW):
            ssum = ssum + psv[t, pl.ds(sbase, L)]
            scnt = scnt + pcv[t, pl.ds(sbase, L)]
        mean = ssum / jnp.maximum(scnt, onev)
        rbuf[pl.ds(sbase, L)] = w / (mean + epsv)
        return carry

    lax.fori_loop(0, S // L, r_body, 0)

    def chunk_body(k, carry):
        base = (wid + k * NW) * CHB
        pltpu.sync_copy(pos_hbm.at[:, pl.ds(base, CHB)], buf)
        pltpu.sync_copy(batch_hbm.at[pl.ds(base, CHB)], bbuf)

        def vec_body(v, c2):
            o = v * L
            b = bbuf[pl.ds(o, L)]
            r = plsc.load_gather(rbuf, [b])
            obuf[0, pl.ds(o, L)] = buf[0, pl.ds(o, L)] * r
            obuf[1, pl.ds(o, L)] = buf[1, pl.ds(o, L)] * r
            obuf[2, pl.ds(o, L)] = buf[2, pl.ds(o, L)] * r
            return c2

        lax.fori_loop(0, VPC, vec_body, 0)
        pltpu.sync_copy(obuf, out_hbm.at[:, pl.ds(base, CHB)])
        return carry

    lax.fori_loop(0, _n_chunks(wid), chunk_body, 0)


def kernel(pos, batch, weight):
    pos_t = jnp.swapaxes(pos, 0, 1)
    wvec = jnp.broadcast_to(weight.reshape(1), (L,)).astype(jnp.float32)
    psum, pcnt = _pass1(pos_t, batch)
    out_t = _pass2(pos_t, batch, wvec, psum, pcnt)
    return jnp.swapaxes(out_t, 0, 1)


# speedup vs baseline: 130.7690x; 2.1749x over previous
import functools

import jax
import jax.numpy as jnp
from jax import lax
from jax.experimental import pallas as pl
from jax.experimental.pallas import tpu as pltpu
from jax.experimental.pallas import tpu_sc as plsc

N = 3200000
S = 1024
EPS = 1e-5
L = 16
NC, NS = 2, 16
NW = NC * NS
CHB = 3200
NCHT = N // CHB
VPC = CHB // L
NFULL = NCHT // NW
NEXTRA = NCHT % NW
SB = 256

_mesh = plsc.VectorSubcoreMesh(core_axis_name="c", subcore_axis_name="s")
_params = pltpu.CompilerParams(needs_layout_passes=False)


def _fast_norm(n2):
    i = lax.bitcast_convert_type(n2, jnp.int32)
    i = jnp.full((L,), 0x5F3759DF, jnp.int32) - lax.shift_right_logical(i, 1)
    y = lax.bitcast_convert_type(i, jnp.float32)
    ah = n2 * jnp.full((L,), 0.5, jnp.float32)
    c15 = jnp.full((L,), 1.5, jnp.float32)
    y = y * (c15 - ah * y * y)
    y = y * (c15 - ah * y * y)
    return n2 * y


@functools.partial(
    pl.kernel,
    mesh=_mesh,
    out_type=[
        jax.ShapeDtypeStruct((NW, S), jnp.float32),
        jax.ShapeDtypeStruct((NW, S), jnp.float32),
    ],
    scratch_types=[
        pltpu.VMEM((3, CHB), jnp.float32),
        pltpu.VMEM((3, CHB), jnp.float32),
        pltpu.VMEM((CHB,), jnp.int32),
        pltpu.VMEM((CHB,), jnp.int32),
        pltpu.VMEM((L * S,), jnp.float32),
        pltpu.VMEM((L * S,), jnp.float32),
        pltpu.VMEM((S,), jnp.float32),
        pltpu.VMEM((S,), jnp.float32),
        pltpu.SemaphoreType.DMA,
        pltpu.SemaphoreType.DMA,
        pltpu.SemaphoreType.DMA,
        pltpu.SemaphoreType.DMA,
    ],
    compiler_params=_params,
)
def _pass1(pos_hbm, batch_hbm, psum_hbm, pcnt_hbm,
           bufa, bufb, bba, bbb, accs, accc, reds, redc,
           spa, sba, spb, sbb):
    wid = lax.axis_index("s") * NC + lax.axis_index("c")
    lanes = lax.iota(jnp.int32, L)
    laneoff = lanes * S
    zero = jnp.zeros((L,), jnp.float32)
    ones = jnp.ones((L,), jnp.float32)
    n_my = jnp.int32(NFULL) + jnp.where(wid < NEXTRA, 1, 0).astype(jnp.int32)

    @plsc.parallel_loop(0, S, unroll=8)
    def _(i):
        accs[pl.ds(i * L, L)] = zero
        accc[pl.ds(i * L, L)] = zero

    def start(k, bufp, bufb2, semp, semb):
        base = (wid + k * NW) * CHB
        pltpu.async_copy(pos_hbm.at[:, pl.ds(base, CHB)], bufp, semp)
        pltpu.async_copy(batch_hbm.at[pl.ds(base, CHB)], bufb2, semb)

    def wait(bufp, bufb2, semp, semb):
        pltpu.make_async_copy(pos_hbm.at[:, pl.ds(0, CHB)], bufp, semp).wait()
        pltpu.make_async_copy(batch_hbm.at[pl.ds(0, CHB)], bufb2, semb).wait()

    def compute(bufp, bufb2):
        @plsc.parallel_loop(0, VPC, unroll=8)
        def _(v):
            o = v * L
            b = bufb2[pl.ds(o, L)]
            x = bufp[0, pl.ds(o, L)]
            y = bufp[1, pl.ds(o, L)]
            z = bufp[2, pl.ds(o, L)]
            nrm = _fast_norm(x * x + y * y + z * z)
            idx = b + laneoff
            plsc.addupdate_scatter(accs, [idx], nrm)
            plsc.addupdate_scatter(accc, [idx], ones)

    start(0, bufa, bba, spa, sba)

    def pair_body(j, carry):
        k1 = 2 * j + 1
        k2 = 2 * j + 2

        @pl.when(k1 < n_my)
        def _():
            start(k1, bufb, bbb, spb, sbb)

        wait(bufa, bba, spa, sba)
        compute(bufa, bba)

        @pl.when(k2 < n_my)
        def _():
            start(k2, bufa, bba, spa, sba)

        @pl.when(k1 < n_my)
        def _():
            wait(bufb, bbb, spb, sbb)
            compute(bufb, bbb)

        return carry

    lax.fori_loop(0, (NFULL + 1) // 2, pair_body, 0)

    def red_body(g, carry):
        sbase = g * L
        ssum = accs[pl.ds(sbase, L)]
        scnt = accc[pl.ds(sbase, L)]
        for c in range(1, L):
            ssum = ssum + accs[pl.ds(c * S + sbase, L)]
            scnt = scnt + accc[pl.ds(c * S + sbase, L)]
        reds[pl.ds(sbase, L)] = ssum
        redc[pl.ds(sbase, L)] = scnt
        return carry

    lax.fori_loop(0, S // L, red_body, 0)
    pltpu.sync_copy(reds, psum_hbm.at[wid])
    pltpu.sync_copy(redc, pcnt_hbm.at[wid])


@functools.partial(
    pl.kernel,
    mesh=_mesh,
    out_type=jax.ShapeDtypeStruct((3, N), jnp.float32),
    scratch_types=[
        pltpu.VMEM((NW, SB), jnp.float32),
        pltpu.VMEM((NW, SB), jnp.float32),
        pltpu.VMEM((S,), jnp.float32),
        pltpu.VMEM((L,), jnp.float32),
        pltpu.VMEM((3, CHB), jnp.float32),
        pltpu.VMEM((3, CHB), jnp.float32),
        pltpu.VMEM((CHB,), jnp.int32),
        pltpu.VMEM((CHB,), jnp.int32),
        pltpu.VMEM((3, CHB), jnp.float32),
        pltpu.VMEM((3, CHB), jnp.float32),
        pltpu.SemaphoreType.DMA,
        pltpu.SemaphoreType.DMA,
        pltpu.SemaphoreType.DMA,
        pltpu.SemaphoreType.DMA,
        pltpu.SemaphoreType.DMA,
        pltpu.SemaphoreType.DMA,
    ],
    compiler_params=_params,
)
def _pass2(pos_hbm, batch_hbm, w_hbm, psum_hbm, pcnt_hbm, out_hbm,
           psb, pcb, rbuf, wbuf, bufa, bufb, bba, bbb, oba, obb,
           spa, sba, spb, sbb, soa, sob):
    wid = lax.axis_index("s") * NC + lax.axis_index("c")
    onev = jnp.ones((L,), jnp.float32)
    epsv = jnp.full((L,), EPS, jnp.float32)
    n_my = jnp.int32(NFULL) + jnp.where(wid < NEXTRA, 1, 0).astype(jnp.int32)

    pltpu.sync_copy(w_hbm, wbuf)
    w = wbuf[pl.ds(0, L)]

    for c in range(S // SB):
        pltpu.sync_copy(psum_hbm.at[:, pl.ds(c * SB, SB)], psb)
        pltpu.sync_copy(pcnt_hbm.at[:, pl.ds(c * SB, SB)], pcb)

        def r_body(g, carry, c=c):
            sbase = g * L
            ssum = psb[0, pl.ds(sbase, L)]
            scnt = pcb[0, pl.ds(sbase, L)]
            for t in range(1, NW):
                ssum = ssum + psb[t, pl.ds(sbase, L)]
                scnt = scnt + pcb[t, pl.ds(sbase, L)]
            mean = ssum / jnp.maximum(scnt, onev)
            rbuf[pl.ds(c * SB + sbase, L)] = w / (mean + epsv)
            return carry

        lax.fori_loop(0, SB // L, r_body, 0)

    def start(k, bufp, bufb2, semp, semb):
        base = (wid + k * NW) * CHB
        pltpu.async_copy(pos_hbm.at[:, pl.ds(base, CHB)], bufp, semp)
        pltpu.async_copy(batch_hbm.at[pl.ds(base, CHB)], bufb2, semb)

    def wait_in(bufp, bufb2, semp, semb):
        pltpu.make_async_copy(pos_hbm.at[:, pl.ds(0, CHB)], bufp, semp).wait()
        pltpu.make_async_copy(batch_hbm.at[pl.ds(0, CHB)], bufb2, semb).wait()

    def start_out(k, obuf, semo):
        base = (wid + k * NW) * CHB
        pltpu.async_copy(obuf, out_hbm.at[:, pl.ds(base, CHB)], semo)

    def wait_out(obuf, semo):
        pltpu.make_async_copy(obuf, out_hbm.at[:, pl.ds(0, CHB)], semo).wait()

    def compute(bufp, bufb2, obuf):
        @plsc.parallel_loop(0, VPC, unroll=8)
        def _(v):
            o = v * L
            b = bufb2[pl.ds(o, L)]
            r = plsc.load_gather(rbuf, [b])
            obuf[0, pl.ds(o, L)] = bufp[0, pl.ds(o, L)] * r
            obuf[1, pl.ds(o, L)] = bufp[1, pl.ds(o, L)] * r
            obuf[2, pl.ds(o, L)] = bufp[2, pl.ds(o, L)] * r

    start(0, bufa, bba, spa, sba)

    def pair_body(j, carry):
        k1 = 2 * j + 1
        k2 = 2 * j + 2

        @pl.when(k1 < n_my)
        def _():
            start(k1, bufb, bbb, spb, sbb)

        wait_in(bufa, bba, spa, sba)

        @pl.when(j > 0)
        def _():
            wait_out(oba, soa)

        compute(bufa, bba, oba)
        start_out(2 * j, oba, soa)

        @pl.when(k2 < n_my)
        def _():
            start(k2, bufa, bba, spa, sba)

        @pl.when(k1 < n_my)
        def _():
            wait_in(bufb, bbb, spb, sbb)

            @pl.when(j > 0)
            def _():
                wait_out(obb, sob)

            compute(bufb, bbb, obb)
            start_out(k1, obb, sob)

        return carry

    lax.fori_loop(0, (NFULL + 1) // 2, pair_body, 0)
    wait_out(oba, soa)
    wait_out(obb, sob)


def kernel(pos, batch, weight):
    pos_t = jnp.swapaxes(pos, 0, 1)
    wvec = jnp.broadcast_to(weight.reshape(1), (L,)).astype(jnp.float32)
    psum, pcnt = _pass1(pos_t, batch)
    out_t = _pass2(pos_t, batch, wvec, psum, pcnt)
    return jnp.swapaxes(out_t, 0, 1)


# speedup vs baseline: 245.4104x; 1.8767x over previous
import functools

import jax
import jax.numpy as jnp
from jax import lax
from jax.experimental import pallas as pl
from jax.experimental.pallas import tpu as pltpu
from jax.experimental.pallas import tpu_sc as plsc

N = 3200000
S = 1024
EPS = 1e-5
L = 16
NC, NS = 2, 16
NW = NC * NS
CHB = 3200
NCHT = N // CHB
VPC = CHB // L
NFULL = NCHT // NW
NEXTRA = NCHT % NW
SB = 256
SSTR = S + 1

_mesh = plsc.VectorSubcoreMesh(core_axis_name="c", subcore_axis_name="s")
_params = pltpu.CompilerParams(needs_layout_passes=False)


def _fast_norm(n2):
    i = lax.bitcast_convert_type(n2, jnp.int32)
    i = jnp.full((L,), 0x5F3759DF, jnp.int32) - lax.shift_right_logical(i, 1)
    y = lax.bitcast_convert_type(i, jnp.float32)
    ah = n2 * jnp.full((L,), 0.5, jnp.float32)
    c15 = jnp.full((L,), 1.5, jnp.float32)
    y = y * (c15 - ah * y * y)
    y = y * (c15 - ah * y * y)
    return n2 * y


@functools.partial(
    pl.kernel,
    mesh=_mesh,
    out_type=[
        jax.ShapeDtypeStruct((NW, S), jnp.float32),
        jax.ShapeDtypeStruct((NW, S), jnp.float32),
    ],
    scratch_types=[
        pltpu.VMEM((3, CHB), jnp.float32),
        pltpu.VMEM((3, CHB), jnp.float32),
        pltpu.VMEM((CHB,), jnp.int32),
        pltpu.VMEM((CHB,), jnp.int32),
        pltpu.VMEM((L * SSTR,), jnp.float32),
        pltpu.VMEM((L * SSTR,), jnp.float32),
        pltpu.VMEM((S,), jnp.float32),
        pltpu.VMEM((S,), jnp.float32),
        pltpu.SemaphoreType.DMA,
        pltpu.SemaphoreType.DMA,
        pltpu.SemaphoreType.DMA,
        pltpu.SemaphoreType.DMA,
    ],
    compiler_params=_params,
)
def _pass1(pos_hbm, batch_hbm, psum_hbm, pcnt_hbm,
           bufa, bufb, bba, bbb, accs, accc, reds, redc,
           spa, sba, spb, sbb):
    wid = lax.axis_index("s") * NC + lax.axis_index("c")
    lanes = lax.iota(jnp.int32, L)
    laneoff = lanes * SSTR
    zero = jnp.zeros((L,), jnp.float32)
    ones = jnp.ones((L,), jnp.float32)
    n_my = jnp.int32(NFULL) + jnp.where(wid < NEXTRA, 1, 0).astype(jnp.int32)

    @plsc.parallel_loop(0, SSTR, unroll=5)
    def _(i):
        accs[pl.ds(i * L, L)] = zero
        accc[pl.ds(i * L, L)] = zero

    def start(k, bufp, bufb2, semp, semb):
        base = (wid + k * NW) * CHB
        pltpu.async_copy(pos_hbm.at[:, pl.ds(base, CHB)], bufp, semp)
        pltpu.async_copy(batch_hbm.at[pl.ds(base, CHB)], bufb2, semb)

    def wait(bufp, bufb2, semp, semb):
        pltpu.make_async_copy(pos_hbm.at[:, pl.ds(0, CHB)], bufp, semp).wait()
        pltpu.make_async_copy(batch_hbm.at[pl.ds(0, CHB)], bufb2, semb).wait()

    def compute(bufp, bufb2):
        @plsc.parallel_loop(0, VPC, unroll=8)
        def _(v):
            o = v * L
            b = bufb2[pl.ds(o, L)]
            x = bufp[0, pl.ds(o, L)]
            y = bufp[1, pl.ds(o, L)]
            z = bufp[2, pl.ds(o, L)]
            nrm = _fast_norm(x * x + y * y + z * z)
            idx = b + laneoff
            plsc.addupdate_scatter(accs, [idx], nrm)
            plsc.addupdate_scatter(accc, [idx], ones)

    start(0, bufa, bba, spa, sba)

    def pair_body(j, carry):
        k1 = 2 * j + 1
        k2 = 2 * j + 2

        @pl.when(k1 < n_my)
        def _():
            start(k1, bufb, bbb, spb, sbb)

        wait(bufa, bba, spa, sba)
        compute(bufa, bba)

        @pl.when(k2 < n_my)
        def _():
            start(k2, bufa, bba, spa, sba)

        @pl.when(k1 < n_my)
        def _():
            wait(bufb, bbb, spb, sbb)
            compute(bufb, bbb)

        return carry

    lax.fori_loop(0, (NFULL + 1) // 2, pair_body, 0)

    def red_body(g, carry):
        sbase = g * L
        ssum = accs[pl.ds(sbase, L)]
        scnt = accc[pl.ds(sbase, L)]
        for c in range(1, L):
            ssum = ssum + accs[pl.ds(c * SSTR + sbase, L)]
            scnt = scnt + accc[pl.ds(c * SSTR + sbase, L)]
        reds[pl.ds(sbase, L)] = ssum
        redc[pl.ds(sbase, L)] = scnt
        return carry

    lax.fori_loop(0, S // L, red_body, 0)
    pltpu.sync_copy(reds, psum_hbm.at[wid])
    pltpu.sync_copy(redc, pcnt_hbm.at[wid])


@functools.partial(
    pl.kernel,
    mesh=_mesh,
    out_type=jax.ShapeDtypeStruct((3, N), jnp.float32),
    scratch_types=[
        pltpu.VMEM((NW, SB), jnp.float32),
        pltpu.VMEM((NW, SB), jnp.float32),
        pltpu.VMEM((S,), jnp.float32),
        pltpu.VMEM((L,), jnp.float32),
        pltpu.VMEM((3, CHB), jnp.float32),
        pltpu.VMEM((3, CHB), jnp.float32),
        pltpu.VMEM((CHB,), jnp.int32),
        pltpu.VMEM((CHB,), jnp.int32),
        pltpu.VMEM((3, CHB), jnp.float32),
        pltpu.VMEM((3, CHB), jnp.float32),
        pltpu.SemaphoreType.DMA,
        pltpu.SemaphoreType.DMA,
        pltpu.SemaphoreType.DMA,
        pltpu.SemaphoreType.DMA,
        pltpu.SemaphoreType.DMA,
        pltpu.SemaphoreType.DMA,
    ],
    compiler_params=_params,
)
def _pass2(pos_hbm, batch_hbm, w_hbm, psum_hbm, pcnt_hbm, out_hbm,
           psb, pcb, rbuf, wbuf, bufa, bufb, bba, bbb, oba, obb,
           spa, sba, spb, sbb, soa, sob):
    wid = lax.axis_index("s") * NC + lax.axis_index("c")
    onev = jnp.ones((L,), jnp.float32)
    epsv = jnp.full((L,), EPS, jnp.float32)
    n_my = jnp.int32(NFULL) + jnp.where(wid < NEXTRA, 1, 0).astype(jnp.int32)

    pltpu.sync_copy(w_hbm, wbuf)
    w = wbuf[pl.ds(0, L)]

    for c in range(S // SB):
        pltpu.sync_copy(psum_hbm.at[:, pl.ds(c * SB, SB)], psb)
        pltpu.sync_copy(pcnt_hbm.at[:, pl.ds(c * SB, SB)], pcb)

        def r_body(g, carry, c=c):
            sbase = g * L
            ssum = psb[0, pl.ds(sbase, L)]
            scnt = pcb[0, pl.ds(sbase, L)]
            for t in range(1, NW):
                ssum = ssum + psb[t, pl.ds(sbase, L)]
                scnt = scnt + pcb[t, pl.ds(sbase, L)]
            mean = ssum / jnp.maximum(scnt, onev)
            rbuf[pl.ds(c * SB + sbase, L)] = w / (mean + epsv)
            return carry

        lax.fori_loop(0, SB // L, r_body, 0)

    def start(k, bufp, bufb2, semp, semb):
        base = (wid + k * NW) * CHB
        pltpu.async_copy(pos_hbm.at[:, pl.ds(base, CHB)], bufp, semp)
        pltpu.async_copy(batch_hbm.at[pl.ds(base, CHB)], bufb2, semb)

    def wait_in(bufp, bufb2, semp, semb):
        pltpu.make_async_copy(pos_hbm.at[:, pl.ds(0, CHB)], bufp, semp).wait()
        pltpu.make_async_copy(batch_hbm.at[pl.ds(0, CHB)], bufb2, semb).wait()

    def start_out(k, obuf, semo):
        base = (wid + k * NW) * CHB
        pltpu.async_copy(obuf, out_hbm.at[:, pl.ds(base, CHB)], semo)

    def wait_out(obuf, semo):
        pltpu.make_async_copy(obuf, out_hbm.at[:, pl.ds(0, CHB)], semo).wait()

    def compute(bufp, bufb2, obuf):
        @plsc.parallel_loop(0, VPC, unroll=8)
        def _(v):
            o = v * L
            b = bufb2[pl.ds(o, L)]
            r = plsc.load_gather(rbuf, [b])
            obuf[0, pl.ds(o, L)] = bufp[0, pl.ds(o, L)] * r
            obuf[1, pl.ds(o, L)] = bufp[1, pl.ds(o, L)] * r
            obuf[2, pl.ds(o, L)] = bufp[2, pl.ds(o, L)] * r

    start(0, bufa, bba, spa, sba)

    def pair_body(j, carry):
        k1 = 2 * j + 1
        k2 = 2 * j + 2

        @pl.when(k1 < n_my)
        def _():
            start(k1, bufb, bbb, spb, sbb)

        wait_in(bufa, bba, spa, sba)

        @pl.when(j > 0)
        def _():
            wait_out(oba, soa)

        compute(bufa, bba, oba)
        start_out(2 * j, oba, soa)

        @pl.when(k2 < n_my)
        def _():
            start(k2, bufa, bba, spa, sba)

        @pl.when(k1 < n_my)
        def _():
            wait_in(bufb, bbb, spb, sbb)

            @pl.when(j > 0)
            def _():
                wait_out(obb, sob)

            compute(bufb, bbb, obb)
            start_out(k1, obb, sob)

        return carry

    lax.fori_loop(0, (NFULL + 1) // 2, pair_body, 0)
    wait_out(oba, soa)
    wait_out(obb, sob)


def kernel(pos, batch, weight):
    pos_t = jnp.swapaxes(pos, 0, 1)
    wvec = jnp.broadcast_to(weight.reshape(1), (L,)).astype(jnp.float32)
    psum, pcnt = _pass1(pos_t, batch)
    out_t = _pass2(pos_t, batch, wvec, psum, pcnt)
    return jnp.swapaxes(out_t, 0, 1)


# speedup vs baseline: 255.5968x; 1.0415x over previous
import functools

import jax
import jax.numpy as jnp
from jax import lax
from jax.experimental import pallas as pl
from jax.experimental.pallas import tpu as pltpu
from jax.experimental.pallas import tpu_sc as plsc

N = 3200000
S = 1024
EPS = 1e-5
L = 16
NC, NS = 2, 16
NW = NC * NS
CHB = 3200
NCHT = N // CHB
VPC = CHB // L
NFULL = NCHT // NW
NEXTRA = NCHT % NW
SB = 256
SSTR = S + 1

_mesh = plsc.VectorSubcoreMesh(core_axis_name="c", subcore_axis_name="s")
_params = pltpu.CompilerParams(needs_layout_passes=False)


def _fast_norm(n2):
    i = lax.bitcast_convert_type(n2, jnp.int32)
    i = jnp.full((L,), 0x5F3759DF, jnp.int32) - lax.shift_right_logical(i, 1)
    y = lax.bitcast_convert_type(i, jnp.float32)
    ah = n2 * jnp.full((L,), 0.5, jnp.float32)
    c15 = jnp.full((L,), 1.5, jnp.float32)
    y = y * (c15 - ah * y * y)
    return n2 * y


@functools.partial(
    pl.kernel,
    mesh=_mesh,
    out_type=[
        jax.ShapeDtypeStruct((NW, S), jnp.float32),
        jax.ShapeDtypeStruct((NW, S), jnp.float32),
    ],
    scratch_types=[
        pltpu.VMEM((3, CHB), jnp.float32),
        pltpu.VMEM((3, CHB), jnp.float32),
        pltpu.VMEM((CHB,), jnp.int32),
        pltpu.VMEM((CHB,), jnp.int32),
        pltpu.VMEM((L * SSTR,), jnp.float32),
        pltpu.VMEM((L * SSTR,), jnp.float32),
        pltpu.VMEM((S,), jnp.float32),
        pltpu.VMEM((S,), jnp.float32),
        pltpu.SemaphoreType.DMA,
        pltpu.SemaphoreType.DMA,
        pltpu.SemaphoreType.DMA,
        pltpu.SemaphoreType.DMA,
    ],
    compiler_params=_params,
)
def _pass1(pos_hbm, batch_hbm, psum_hbm, pcnt_hbm,
           bufa, bufb, bba, bbb, accs, accc, reds, redc,
           spa, sba, spb, sbb):
    wid = lax.axis_index("s") * NC + lax.axis_index("c")
    lanes = lax.iota(jnp.int32, L)
    laneoff = lanes * SSTR
    zero = jnp.zeros((L,), jnp.float32)
    ones = jnp.ones((L,), jnp.float32)
    n_my = jnp.int32(NFULL) + jnp.where(wid < NEXTRA, 1, 0).astype(jnp.int32)

    @plsc.parallel_loop(0, SSTR, unroll=5)
    def _(i):
        accs[pl.ds(i * L, L)] = zero
        accc[pl.ds(i * L, L)] = zero

    def start(k, bufp, bufb2, semp, semb):
        base = (wid + k * NW) * CHB
        pltpu.async_copy(pos_hbm.at[:, pl.ds(base, CHB)], bufp, semp)
        pltpu.async_copy(batch_hbm.at[pl.ds(base, CHB)], bufb2, semb)

    def wait(bufp, bufb2, semp, semb):
        pltpu.make_async_copy(pos_hbm.at[:, pl.ds(0, CHB)], bufp, semp).wait()
        pltpu.make_async_copy(batch_hbm.at[pl.ds(0, CHB)], bufb2, semb).wait()

    def compute(bufp, bufb2):
        @plsc.parallel_loop(0, VPC, unroll=16)
        def _(v):
            o = v * L
            b = bufb2[pl.ds(o, L)]
            x = bufp[0, pl.ds(o, L)]
            y = bufp[1, pl.ds(o, L)]
            z = bufp[2, pl.ds(o, L)]
            nrm = _fast_norm(x * x + y * y + z * z)
            idx = b + laneoff
            plsc.addupdate_scatter(accs, [idx], nrm)
            plsc.addupdate_scatter(accc, [idx], ones)

    start(0, bufa, bba, spa, sba)

    def pair_body(j, carry):
        k1 = 2 * j + 1
        k2 = 2 * j + 2

        @pl.when(k1 < n_my)
        def _():
            start(k1, bufb, bbb, spb, sbb)

        wait(bufa, bba, spa, sba)
        compute(bufa, bba)

        @pl.when(k2 < n_my)
        def _():
            start(k2, bufa, bba, spa, sba)

        @pl.when(k1 < n_my)
        def _():
            wait(bufb, bbb, spb, sbb)
            compute(bufb, bbb)

        return carry

    lax.fori_loop(0, (NFULL + 1) // 2, pair_body, 0)

    def red_body(g, carry):
        sbase = g * L
        ssum = accs[pl.ds(sbase, L)]
        scnt = accc[pl.ds(sbase, L)]
        for c in range(1, L):
            ssum = ssum + accs[pl.ds(c * SSTR + sbase, L)]
            scnt = scnt + accc[pl.ds(c * SSTR + sbase, L)]
        reds[pl.ds(sbase, L)] = ssum
        redc[pl.ds(sbase, L)] = scnt
        return carry

    lax.fori_loop(0, S // L, red_body, 0)
    pltpu.sync_copy(reds, psum_hbm.at[wid])
    pltpu.sync_copy(redc, pcnt_hbm.at[wid])


@functools.partial(
    pl.kernel,
    mesh=_mesh,
    out_type=jax.ShapeDtypeStruct((3, N), jnp.float32),
    scratch_types=[
        pltpu.VMEM((NW, S), jnp.float32),
        pltpu.VMEM((NW, S), jnp.float32),
        pltpu.VMEM((S,), jnp.float32),
        pltpu.VMEM((L,), jnp.float32),
        pltpu.VMEM((3, CHB), jnp.float32),
        pltpu.VMEM((3, CHB), jnp.float32),
        pltpu.VMEM((CHB,), jnp.int32),
        pltpu.VMEM((CHB,), jnp.int32),
        pltpu.VMEM((3, CHB), jnp.float32),
        pltpu.VMEM((3, CHB), jnp.float32),
        pltpu.SemaphoreType.DMA,
        pltpu.SemaphoreType.DMA,
        pltpu.SemaphoreType.DMA,
        pltpu.SemaphoreType.DMA,
        pltpu.SemaphoreType.DMA,
        pltpu.SemaphoreType.DMA,
        pltpu.SemaphoreType.DMA,
        pltpu.SemaphoreType.DMA,
    ],
    compiler_params=_params,
)
def _pass2(pos_hbm, batch_hbm, w_hbm, psum_hbm, pcnt_hbm, out_hbm,
           psb, pcb, rbuf, wbuf, bufa, bufb, bba, bbb, oba, obb,
           spa, sba, spb, sbb, soa, sob, sps, spc):
    wid = lax.axis_index("s") * NC + lax.axis_index("c")
    onev = jnp.ones((L,), jnp.float32)
    epsv = jnp.full((L,), EPS, jnp.float32)
    n_my = jnp.int32(NFULL) + jnp.where(wid < NEXTRA, 1, 0).astype(jnp.int32)

    pltpu.async_copy(psum_hbm, psb, sps)
    pltpu.async_copy(pcnt_hbm, pcb, spc)
    pltpu.sync_copy(w_hbm, wbuf)
    w = wbuf[pl.ds(0, L)]

    def start(k, bufp, bufb2, semp, semb):
        base = (wid + k * NW) * CHB
        pltpu.async_copy(pos_hbm.at[:, pl.ds(base, CHB)], bufp, semp)
        pltpu.async_copy(batch_hbm.at[pl.ds(base, CHB)], bufb2, semb)

    def wait_in(bufp, bufb2, semp, semb):
        pltpu.make_async_copy(pos_hbm.at[:, pl.ds(0, CHB)], bufp, semp).wait()
        pltpu.make_async_copy(batch_hbm.at[pl.ds(0, CHB)], bufb2, semb).wait()

    def start_out(k, obuf, semo):
        base = (wid + k * NW) * CHB
        pltpu.async_copy(obuf, out_hbm.at[:, pl.ds(base, CHB)], semo)

    def wait_out(obuf, semo):
        pltpu.make_async_copy(obuf, out_hbm.at[:, pl.ds(0, CHB)], semo).wait()

    def compute(bufp, bufb2, obuf):
        @plsc.parallel_loop(0, VPC, unroll=16)
        def _(v):
            o = v * L
            b = bufb2[pl.ds(o, L)]
            r = plsc.load_gather(rbuf, [b])
            obuf[0, pl.ds(o, L)] = bufp[0, pl.ds(o, L)] * r
            obuf[1, pl.ds(o, L)] = bufp[1, pl.ds(o, L)] * r
            obuf[2, pl.ds(o, L)] = bufp[2, pl.ds(o, L)] * r

    start(0, bufa, bba, spa, sba)

    pltpu.make_async_copy(psum_hbm, psb, sps).wait()
    pltpu.make_async_copy(pcnt_hbm, pcb, spc).wait()

    def r_body(g, carry):
        sbase = g * L
        ssum = psb[0, pl.ds(sbase, L)]
        scnt = pcb[0, pl.ds(sbase, L)]
        for t in range(1, NW):
            ssum = ssum + psb[t, pl.ds(sbase, L)]
            scnt = scnt + pcb[t, pl.ds(sbase, L)]
        mean = ssum / jnp.maximum(scnt, onev)
        rbuf[pl.ds(sbase, L)] = w / (mean + epsv)
        return carry

    lax.fori_loop(0, S // L, r_body, 0)

    def pair_body(j, carry):
        k1 = 2 * j + 1
        k2 = 2 * j + 2

        @pl.when(k1 < n_my)
        def _():
            start(k1, bufb, bbb, spb, sbb)

        wait_in(bufa, bba, spa, sba)

        @pl.when(j > 0)
        def _():
            wait_out(oba, soa)

        compute(bufa, bba, oba)
        start_out(2 * j, oba, soa)

        @pl.when(k2 < n_my)
        def _():
            start(k2, bufa, bba, spa, sba)

        @pl.when(k1 < n_my)
        def _():
            wait_in(bufb, bbb, spb, sbb)

            @pl.when(j > 0)
            def _():
                wait_out(obb, sob)

            compute(bufb, bbb, obb)
            start_out(k1, obb, sob)

        return carry

    lax.fori_loop(0, (NFULL + 1) // 2, pair_body, 0)
    wait_out(oba, soa)
    wait_out(obb, sob)


def kernel(pos, batch, weight):
    pos_t = jnp.swapaxes(pos, 0, 1)
    wvec = jnp.broadcast_to(weight.reshape(1), (L,)).astype(jnp.float32)
    psum, pcnt = _pass1(pos_t, batch)
    out_t = _pass2(pos_t, batch, wvec, psum, pcnt)
    return jnp.swapaxes(out_t, 0, 1)
